# baseline (speedup 1.0000x reference)
import functools

import jax
import jax.numpy as jnp
from jax import lax
from jax.experimental import pallas as pl
from jax.experimental.pallas import tpu as pltpu
from jax.experimental.pallas import tpu_sc as plsc

N = 4096
LAT = 128
HEADS = 2
DH = LAT // HEADS
TASKS = 8
BLK = 512
NBLK = N // BLK
BLKG = 2048
DHE = DH + 8
LOG2E = 1.4426950408889634
CH = 1024
NCH = N // CH
NEG = -1e30

_SC_CORES = 2
_SC_SUBCORES = 16
_NW = _SC_CORES * _SC_SUBCORES
BPW = N // _NW


def _full(shape):
    return pl.BlockSpec(shape, lambda i: tuple(0 for _ in shape))


def _rows(shape):
    return pl.BlockSpec(shape, lambda i: (i,) + tuple(0 for _ in shape[1:]))


_SMEM = pl.BlockSpec(memory_space=pltpu.SMEM)
_PARALLEL = pltpu.CompilerParams(dimension_semantics=("parallel",))


def _sc_permute(src, idx2d, direction):
    mesh = plsc.VectorSubcoreMesh(core_axis_name="c", subcore_axis_name="s",
                                  num_cores=_SC_CORES,
                                  num_subcores=_SC_SUBCORES)

    @functools.partial(
        pl.kernel, mesh=mesh,
        out_type=jax.ShapeDtypeStruct((N, LAT), jnp.float32),
        scratch_types=[pltpu.VMEM((BPW,), jnp.int32),
                       pltpu.VMEM((BPW, LAT), jnp.float32),
                       pltpu.SemaphoreType.DMA],
    )
    def k(src_hbm, idx_hbm, out_hbm, idx_v, rows_v, sem):
        wid = lax.axis_index("s") * _SC_CORES + lax.axis_index("c")
        base = wid * BPW
        pltpu.sync_copy(idx_hbm.at[wid], idx_v)
        if direction == "scatter":
            pltpu.sync_copy(src_hbm.at[pl.ds(base, BPW)], rows_v)
            pltpu.async_copy(rows_v, out_hbm.at[idx_v], sem).wait()
        else:
            pltpu.async_copy(src_hbm.at[idx_v], rows_v, sem).wait()
            pltpu.sync_copy(rows_v, out_hbm.at[pl.ds(base, BPW)])

    return k(src, idx2d)


def _set_mlp_body(xb, yb, w0x, w0y, b0, w1, b1, out):
    h = (jnp.dot(xb[...], w0x[...], preferred_element_type=jnp.float32)
         + jnp.dot(yb[...], w0y[...], preferred_element_type=jnp.float32)
         + b0[...])
    h = jnp.maximum(h, 0.0)
    out[...] = jnp.dot(h, w1[...], preferred_element_type=jnp.float32) + b1[...]


def _set_mlp(x, y, w0x, w0y, b0, w1, b1):
    return pl.pallas_call(
        _set_mlp_body,
        grid=(NBLK,),
        in_specs=[_rows((BLK, x.shape[1])), _rows((BLK, y.shape[1])),
                  _full(w0x.shape), _full(w0y.shape), _full((1, LAT)),
                  _full((LAT, LAT)), _full((1, LAT))],
        out_specs=_rows((BLK, LAT)),
        out_shape=jax.ShapeDtypeStruct((N, LAT), jnp.float32),
        compiler_params=_PARALLEL,
    )(x, y, w0x, w0y, b0, w1, b1)


def _pt_qkv_body(tfl, tfh, sb, labb, wq, wk, wv, qo, kto, vo, kacc, vacc):
    b = pl.program_id(0)
    tl = tfl[b]
    th = tfh[b]
    s = sb[...]
    lab = labb[...]
    oh = (lab == jax.lax.broadcasted_iota(jnp.int32, (1, TASKS), 1)
          ).astype(jnp.float32)
    qo[...] = jnp.zeros((BLK, LAT), jnp.float32)
    kacc[...] = jnp.zeros((BLK, LAT), jnp.float32)
    vacc[...] = jnp.zeros((BLK, LAT), jnp.float32)
    for t in range(TASKS):
        @pl.when((t >= tl) & (t <= th))
        def _(t=t):
            m = oh[:, t:t + 1]
            qo[...] += m * jnp.dot(s, wq[t], preferred_element_type=jnp.float32)
            kacc[...] += m * jnp.dot(s, wk[t], preferred_element_type=jnp.float32)
            vacc[...] += m * jnp.dot(s, wv[t], preferred_element_type=jnp.float32)
    kto[0] = kacc[...].T
    ones = jnp.ones((BLK, 8), jnp.float32)
    v = vacc[...]
    vo[0] = jnp.concatenate([v[:, :DH], ones], axis=1)
    vo[1] = jnp.concatenate([v[:, DH:], ones], axis=1)


def _pt_qkv(s, lab_col, tfl, tfh, wq, wk, wv):
    out = jax.ShapeDtypeStruct((N, LAT), jnp.float32)
    out_kt = jax.ShapeDtypeStruct((NCH, LAT, CH), jnp.float32)
    out_v = jax.ShapeDtypeStruct((HEADS, N, DHE), jnp.float32)
    cpb = CH // BLK
    return pl.pallas_call(
        _pt_qkv_body,
        grid=(NBLK,),
        in_specs=[_SMEM, _SMEM, _rows((BLK, LAT)), _rows((BLK, 1)),
                  _full((TASKS, LAT, LAT)), _full((TASKS, LAT, LAT)),
                  _full((TASKS, LAT, LAT))],
        out_specs=[_rows((BLK, LAT)),
                   pl.BlockSpec((1, LAT, BLK),
                                lambda i: (i // cpb, 0, i % cpb)),
                   pl.BlockSpec((HEADS, BLK, DHE), lambda i: (0, i, 0))],
        out_shape=[out, out_kt, out_v],
        scratch_shapes=[pltpu.VMEM((BLK, LAT), jnp.float32),
                        pltpu.VMEM((BLK, LAT), jnp.float32)],
        compiler_params=_PARALLEL,
    )(tfl, tfh, s, lab_col, wq, wk, wv)


def _pt_attn_body(tfl, tfh, blo, bhi, sb, qb, labb, labch, kf, vf,
                  wo, gamma, beta, out, acc_ref, m_ref, l_ref, proj_ref):
    b = pl.program_id(0)
    lo = blo[b]
    hi = bhi[b]
    tl = tfl[b]
    th = tfh[b]
    q = qb[...]
    lab = labb[...]
    j0 = b // (CH // BLK)
    madd0 = jnp.where(lab == labch[j0], 0.0, NEG)
    kt0 = kf[j0]
    for h in range(HEADS):
        qh = q[:, h * DH:(h + 1) * DH] * (0.125 * LOG2E)
        logits = lax.dot_general(
            qh, kt0[h * DH:(h + 1) * DH, :], (((1,), (0,)), ((), ())),
            preferred_element_type=jnp.float32) + madd0
        m = jnp.max(logits, 1, keepdims=True)
        p = jnp.exp2(logits - m)
        m_ref[:, h:h + 1] = m
        avd = jnp.dot(p, vf[h, pl.ds(j0 * CH, CH), :],
                      preferred_element_type=jnp.float32)
        l_ref[:, h:h + 1] = avd[:, DH:DH + 1]
        acc_ref[:, h * DH:(h + 1) * DH] = avd[:, :DH]
    for j in range(NCH):
        @pl.when((j >= lo) & (j <= hi) & (j != j0))
        def _(j=j):
            madd = jnp.where(lab == labch[j], 0.0, NEG)
            for h in range(HEADS):
                qh = q[:, h * DH:(h + 1) * DH] * (0.125 * LOG2E)
                kth = kf[j, h * DH:(h + 1) * DH, :]
                logits = lax.dot_general(
                    qh, kth, (((1,), (0,)), ((), ())),
                    preferred_element_type=jnp.float32) + madd
                mprev = m_ref[:, h:h + 1]
                mnew = jnp.maximum(mprev, jnp.max(logits, 1, keepdims=True))
                p = jnp.exp2(logits - mnew)
                scale = jnp.exp2(mprev - mnew)
                avd = jnp.dot(p, vf[h, j * CH:(j + 1) * CH, :],
                              preferred_element_type=jnp.float32)
                l_ref[:, h:h + 1] = (l_ref[:, h:h + 1] * scale
                                     + avd[:, DH:DH + 1])
                acc_ref[:, h * DH:(h + 1) * DH] = (
                    acc_ref[:, h * DH:(h + 1) * DH] * scale + avd[:, :DH])
                m_ref[:, h:h + 1] = mnew
    denom = jnp.concatenate(
        [jnp.broadcast_to(l_ref[:, h:h + 1], (BLK, DH)) for h in range(HEADS)],
        axis=1)
    o = acc_ref[...] / denom
    oh = (lab == jax.lax.broadcasted_iota(jnp.int32, (1, TASKS), 1)
          ).astype(jnp.float32)
    proj_ref[...] = jnp.zeros((BLK, LAT), jnp.float32)
    for t in range(TASKS):
        @pl.when((t >= tl) & (t <= th))
        def _(t=t):
            proj_ref[...] += oh[:, t:t + 1] * jnp.dot(
                o, wo[t], preferred_element_type=jnp.float32)
    gamma_b = jnp.dot(oh, gamma[...], preferred_element_type=jnp.float32)
    beta_b = jnp.dot(oh, beta[...], preferred_element_type=jnp.float32)
    hr = sb[...] + proj_ref[...]
    mu = jnp.mean(hr, axis=1, keepdims=True)
    var = jnp.mean((hr - mu) ** 2, axis=1, keepdims=True)
    out[...] = (hr - mu) * lax.rsqrt(var + 1e-5) * gamma_b + beta_b


def _pt_attn_layer(s, lab_col, lab_chunks, tfl, tfh, blo, bhi,
                   wq, wk, wv, wo, gamma, beta):
    q, k, v = _pt_qkv(s, lab_col, tfl, tfh, wq, wk, wv)
    return pl.pallas_call(
        _pt_attn_body,
        grid=(NBLK,),
        in_specs=[_SMEM, _SMEM, _SMEM, _SMEM,
                  _rows((BLK, LAT)), _rows((BLK, LAT)), _rows((BLK, 1)),
                  _full((NCH, 1, CH)), _full((NCH, LAT, CH)),
                  _full((HEADS, N, DHE)),
                  _full((TASKS, LAT, LAT)), _full((TASKS, LAT)),
                  _full((TASKS, LAT))],
        out_specs=_rows((BLK, LAT)),
        out_shape=jax.ShapeDtypeStruct((N, LAT), jnp.float32),
        scratch_shapes=[pltpu.VMEM((BLK, LAT), jnp.float32),
                        pltpu.VMEM((BLK, HEADS), jnp.float32),
                        pltpu.VMEM((BLK, HEADS), jnp.float32),
                        pltpu.VMEM((BLK, LAT), jnp.float32)],
        compiler_params=_PARALLEL,
    )(tfl, tfh, blo, bhi, s, q, lab_col, lab_chunks, k, v, wo, gamma, beta)


def _g_qkv_body(sb, wq, wk, wv, qo, kto, vo):
    s = sb[...]
    qo[...] = jnp.dot(s, wq[...], preferred_element_type=jnp.float32)
    kto[...] = jnp.dot(s, wk[...], preferred_element_type=jnp.float32).T
    v = jnp.dot(s, wv[...], preferred_element_type=jnp.float32)
    ones = jnp.ones((BLK, 8), jnp.float32)
    vo[0] = jnp.concatenate([v[:, :DH], ones], axis=1)
    vo[1] = jnp.concatenate([v[:, DH:], ones], axis=1)


def _g_attn_body(sb, qb, ktf, vf, wo, gamma, beta, out):
    q = qb[...]
    kt = ktf[...]
    outs = []
    for h in range(HEADS):
        qh = q[:, h * DH:(h + 1) * DH] * (0.125 * LOG2E)
        kth = kt[h * DH:(h + 1) * DH, :]
        logits = lax.dot_general(
            qh, kth, (((1,), (0,)), ((), ())),
            preferred_element_type=jnp.float32)
        m = jnp.max(logits, axis=1, keepdims=True)
        e = jnp.exp2(logits - m)
        avd = jnp.dot(e, vf[h], preferred_element_type=jnp.float32)
        outs.append(avd[:, :DH] / avd[:, DH:DH + 1])
    o = jnp.concatenate(outs, axis=1)
    proj = jnp.dot(o, wo[...], preferred_element_type=jnp.float32)
    hr = sb[...] + proj
    mu = jnp.mean(hr, axis=1, keepdims=True)
    var = jnp.mean((hr - mu) ** 2, axis=1, keepdims=True)
    out[...] = (hr - mu) * lax.rsqrt(var + 1e-5) * gamma[...] + beta[...]


def _g_attn_layer(s, wq, wk, wv, wo, gamma, beta):
    out = jax.ShapeDtypeStruct((N, LAT), jnp.float32)
    out_kt = jax.ShapeDtypeStruct((LAT, N), jnp.float32)
    out_v = jax.ShapeDtypeStruct((HEADS, N, DHE), jnp.float32)
    q, kt, v = pl.pallas_call(
        _g_qkv_body,
        grid=(NBLK,),
        in_specs=[_rows((BLK, LAT))] + [_full((LAT, LAT))] * 3,
        out_specs=[_rows((BLK, LAT)),
                   pl.BlockSpec((LAT, BLK), lambda i: (0, i)),
                   pl.BlockSpec((HEADS, BLK, DHE), lambda i: (0, i, 0))],
        out_shape=[out, out_kt, out_v],
        compiler_params=_PARALLEL,
    )(s, wq, wk, wv)
    return pl.pallas_call(
        _g_attn_body,
        grid=(N // BLKG,),
        in_specs=[_rows((BLKG, LAT)), _rows((BLKG, LAT)),
                  _full((LAT, N)), _full((HEADS, N, DHE)),
                  _full((LAT, LAT)), _full((1, LAT)), _full((1, LAT))],
        out_specs=_rows((BLKG, LAT)),
        out_shape=out,
        compiler_params=_PARALLEL,
    )(s, q, kt, v, wo, gamma, beta)


def _head_body(tf, w0, b0, w1, b1, wmu, bmu, wsig, bsig, muo, sigo):
    m = jnp.mean(tf[...], axis=0, keepdims=True)
    h = jnp.maximum(jnp.dot(m, w0[...], preferred_element_type=jnp.float32)
                    + b0[...], 0.0)
    h = jnp.dot(h, w1[...], preferred_element_type=jnp.float32) + b1[...]
    muo[...] = jnp.dot(h, wmu[...], preferred_element_type=jnp.float32) + bmu[...]
    z = jnp.dot(h, wsig[...], preferred_element_type=jnp.float32) + bsig[...]
    sigo[...] = 0.1 + 0.9 * jax.nn.sigmoid(z)


def _head(t, w0, b0, w1, b1, wmu, bmu, wsig, bsig):
    out = jax.ShapeDtypeStruct((1, LAT), jnp.float32)
    return pl.pallas_call(
        _head_body,
        grid=(1,),
        in_specs=[_full((N, LAT))] + [_full((LAT, LAT)), _full((1, LAT))] * 4,
        out_specs=[_full((1, LAT))] * 2,
        out_shape=[out, out],
    )(t, w0, b0, w1, b1, wmu, bmu, wsig, bsig)


def kernel(x, y, task_labels, set_W0, set_b0, set_W1, set_b1,
           pt_Wq, pt_Wk, pt_Wv, pt_Wo, pt_gamma, pt_beta,
           g_Wq, g_Wk, g_Wv, g_Wo, g_gamma, g_beta,
           am_W0, am_b0, am_W1, am_b1, am_Wmu, am_bmu, am_Wsig, am_bsig):
    r = lambda b: b.reshape(1, LAT)

    lab = task_labels.astype(jnp.int32)
    tids = jnp.arange(TASKS, dtype=jnp.int32)
    oh = (lab[:, None] == tids[None, :]).astype(jnp.int32)
    counts = oh.sum(0)
    ends = jnp.cumsum(counts)
    starts = ends - counts
    cc = jnp.cumsum(oh, axis=0)
    rank = ((oh * starts[None, :]).sum(1) + (oh * cc).sum(1) - 1
            ).astype(jnp.int32)
    pos = jnp.arange(N, dtype=jnp.int32)
    lab_sorted = (pos[:, None] >= ends[None, :]).sum(1).astype(jnp.int32)
    lab_col = lab_sorted.reshape(N, 1)
    lab_chunks = lab_sorted.reshape(NCH, 1, CH)
    tfl = lab_sorted[::BLK]
    tfh = lab_sorted[BLK - 1::BLK]
    ohl = (tfl[:, None] == tids[None, :]).astype(jnp.int32)
    ohh = (tfh[:, None] == tids[None, :]).astype(jnp.int32)
    kstart = (ohl * starts[None, :]).sum(1)
    kend = (ohh * ends[None, :]).sum(1)
    blo = (kstart // CH).astype(jnp.int32)
    bhi = ((kend - 1) // CH).astype(jnp.int32)
    idx2d = rank.reshape(_NW, BPW)

    s = _set_mlp(x, y, set_W0[:x.shape[1]], set_W0[x.shape[1]:],
                 r(set_b0), set_W1, r(set_b1))

    sl = _sc_permute(s, idx2d, "scatter")
    for l in range(pt_Wq.shape[1]):
        sl = _pt_attn_layer(sl, lab_col, lab_chunks, tfl, tfh, blo, bhi,
                            pt_Wq[:, l], pt_Wk[:, l], pt_Wv[:, l],
                            pt_Wo[:, l], pt_gamma[:, l], pt_beta[:, l])

    t = sl
    for l in range(g_Wq.shape[0]):
        t = _g_attn_layer(t, g_Wq[l], g_Wk[l], g_Wv[l], g_Wo[l],
                          r(g_gamma[l]), r(g_beta[l]))

    mu, sig = _head(t, am_W0, r(am_b0), am_W1, r(am_b1),
                    am_Wmu, r(am_bmu), am_Wsig, r(am_bsig))
    s_local = _sc_permute(sl, idx2d, "gather")
    temp = _sc_permute(t, idx2d, "gather")
    return mu.reshape(LAT), sig.reshape(LAT), s_local, temp


# speedup vs baseline: 1.1186x; 1.1186x over previous
import functools

import jax
import jax.numpy as jnp
from jax import lax
from jax.experimental import pallas as pl
from jax.experimental.pallas import tpu as pltpu
from jax.experimental.pallas import tpu_sc as plsc

N = 4096
LAT = 128
HEADS = 2
DH = LAT // HEADS
TASKS = 8
BLK = 512
NBLK = N // BLK
BLKG = 512
DHE = DH + 8
LOG2E = 1.4426950408889634
CH = 1024
NCH = N // CH
NEG = -1e30

_SC_CORES = 2
_SC_SUBCORES = 16
_NW = _SC_CORES * _SC_SUBCORES
BPW = N // _NW


def _full(shape):
    return pl.BlockSpec(shape, lambda i: tuple(0 for _ in shape))


def _rows(shape):
    return pl.BlockSpec(shape, lambda i: (i,) + tuple(0 for _ in shape[1:]))


_SMEM = pl.BlockSpec(memory_space=pltpu.SMEM)
_PARALLEL = pltpu.CompilerParams(dimension_semantics=("parallel",))


def _sc_permute(src, idx2d, direction):
    mesh = plsc.VectorSubcoreMesh(core_axis_name="c", subcore_axis_name="s",
                                  num_cores=_SC_CORES,
                                  num_subcores=_SC_SUBCORES)

    @functools.partial(
        pl.kernel, mesh=mesh,
        out_type=jax.ShapeDtypeStruct((N, LAT), jnp.float32),
        scratch_types=[pltpu.VMEM((BPW,), jnp.int32),
                       pltpu.VMEM((BPW, LAT), jnp.float32),
                       pltpu.SemaphoreType.DMA],
    )
    def k(src_hbm, idx_hbm, out_hbm, idx_v, rows_v, sem):
        wid = lax.axis_index("s") * _SC_CORES + lax.axis_index("c")
        base = wid * BPW
        pltpu.sync_copy(idx_hbm.at[wid], idx_v)
        if direction == "scatter":
            pltpu.sync_copy(src_hbm.at[pl.ds(base, BPW)], rows_v)
            pltpu.async_copy(rows_v, out_hbm.at[idx_v], sem).wait()
        else:
            pltpu.async_copy(src_hbm.at[idx_v], rows_v, sem).wait()
            pltpu.sync_copy(rows_v, out_hbm.at[pl.ds(base, BPW)])

    return k(src, idx2d)


def _set_mlp_body(xb, yb, w0x, w0y, b0, w1, b1, out):
    h = (jnp.dot(xb[...], w0x[...], preferred_element_type=jnp.float32)
         + jnp.dot(yb[...], w0y[...], preferred_element_type=jnp.float32)
         + b0[...])
    h = jnp.maximum(h, 0.0)
    out[...] = jnp.dot(h, w1[...], preferred_element_type=jnp.float32) + b1[...]


def _set_mlp(x, y, w0x, w0y, b0, w1, b1):
    return pl.pallas_call(
        _set_mlp_body,
        grid=(NBLK,),
        in_specs=[_rows((BLK, x.shape[1])), _rows((BLK, y.shape[1])),
                  _full(w0x.shape), _full(w0y.shape), _full((1, LAT)),
                  _full((LAT, LAT)), _full((1, LAT))],
        out_specs=_rows((BLK, LAT)),
        out_shape=jax.ShapeDtypeStruct((N, LAT), jnp.float32),
        compiler_params=_PARALLEL,
    )(x, y, w0x, w0y, b0, w1, b1)


def _pt_qkv_body(tfl, tfh, sb, labb, wq, wk, wv, qo, kto, vo, kacc, vacc):
    b = pl.program_id(0)
    tl = tfl[b]
    th = tfh[b]
    s = sb[...]
    lab = labb[...]
    oh = (lab == jax.lax.broadcasted_iota(jnp.int32, (1, TASKS), 1)
          ).astype(jnp.float32)
    qo[...] = jnp.zeros((BLK, LAT), jnp.float32)
    kacc[...] = jnp.zeros((BLK, LAT), jnp.float32)
    vacc[...] = jnp.zeros((BLK, LAT), jnp.float32)
    for t in range(TASKS):
        @pl.when((t >= tl) & (t <= th))
        def _(t=t):
            m = oh[:, t:t + 1]
            qo[...] += m * jnp.dot(s, wq[t], preferred_element_type=jnp.float32)
            kacc[...] += m * jnp.dot(s, wk[t], preferred_element_type=jnp.float32)
            vacc[...] += m * jnp.dot(s, wv[t], preferred_element_type=jnp.float32)
    kto[0] = kacc[...].T
    ones = jnp.ones((BLK, 8), jnp.float32)
    v = vacc[...]
    vo[0] = jnp.concatenate([v[:, :DH], ones], axis=1)
    vo[1] = jnp.concatenate([v[:, DH:], ones], axis=1)


def _pt_qkv(s, lab_col, tfl, tfh, wq, wk, wv):
    out = jax.ShapeDtypeStruct((N, LAT), jnp.float32)
    out_kt = jax.ShapeDtypeStruct((NCH, LAT, CH), jnp.float32)
    out_v = jax.ShapeDtypeStruct((HEADS, N, DHE), jnp.float32)
    cpb = CH // BLK
    return pl.pallas_call(
        _pt_qkv_body,
        grid=(NBLK,),
        in_specs=[_SMEM, _SMEM, _rows((BLK, LAT)), _rows((BLK, 1)),
                  _full((TASKS, LAT, LAT)), _full((TASKS, LAT, LAT)),
                  _full((TASKS, LAT, LAT))],
        out_specs=[_rows((BLK, LAT)),
                   pl.BlockSpec((1, LAT, BLK),
                                lambda i: (i // cpb, 0, i % cpb)),
                   pl.BlockSpec((HEADS, BLK, DHE), lambda i: (0, i, 0))],
        out_shape=[out, out_kt, out_v],
        scratch_shapes=[pltpu.VMEM((BLK, LAT), jnp.float32),
                        pltpu.VMEM((BLK, LAT), jnp.float32)],
        compiler_params=_PARALLEL,
    )(tfl, tfh, s, lab_col, wq, wk, wv)


def _pt_attn_body(tfl, tfh, blo, bhi, sb, qb, labb, labch, kf, vf,
                  wo, gamma, beta, out, acc_ref, m_ref, l_ref, proj_ref):
    b = pl.program_id(0)
    lo = blo[b]
    hi = bhi[b]
    tl = tfl[b]
    th = tfh[b]
    q = qb[...]
    lab = labb[...]
    j0 = b // (CH // BLK)
    madd0 = jnp.where(lab == labch[j0], 0.0, NEG)
    kt0 = kf[j0]
    for h in range(HEADS):
        qh = q[:, h * DH:(h + 1) * DH] * (0.125 * LOG2E)
        logits = lax.dot_general(
            qh, kt0[h * DH:(h + 1) * DH, :], (((1,), (0,)), ((), ())),
            preferred_element_type=jnp.float32) + madd0
        m = jnp.max(logits, 1, keepdims=True)
        p = jnp.exp2(logits - m)
        m_ref[:, h:h + 1] = m
        avd = jnp.dot(p, vf[h, pl.ds(j0 * CH, CH), :],
                      preferred_element_type=jnp.float32)
        l_ref[:, h:h + 1] = avd[:, DH:DH + 1]
        acc_ref[:, h * DH:(h + 1) * DH] = avd[:, :DH]
    for j in range(NCH):
        @pl.when((j >= lo) & (j <= hi) & (j != j0))
        def _(j=j):
            madd = jnp.where(lab == labch[j], 0.0, NEG)
            for h in range(HEADS):
                qh = q[:, h * DH:(h + 1) * DH] * (0.125 * LOG2E)
                kth = kf[j, h * DH:(h + 1) * DH, :]
                logits = lax.dot_general(
                    qh, kth, (((1,), (0,)), ((), ())),
                    preferred_element_type=jnp.float32) + madd
                mprev = m_ref[:, h:h + 1]
                mnew = jnp.maximum(mprev, jnp.max(logits, 1, keepdims=True))
                p = jnp.exp2(logits - mnew)
                scale = jnp.exp2(mprev - mnew)
                avd = jnp.dot(p, vf[h, j * CH:(j + 1) * CH, :],
                              preferred_element_type=jnp.float32)
                l_ref[:, h:h + 1] = (l_ref[:, h:h + 1] * scale
                                     + avd[:, DH:DH + 1])
                acc_ref[:, h * DH:(h + 1) * DH] = (
                    acc_ref[:, h * DH:(h + 1) * DH] * scale + avd[:, :DH])
                m_ref[:, h:h + 1] = mnew
    denom = jnp.concatenate(
        [jnp.broadcast_to(l_ref[:, h:h + 1], (BLK, DH)) for h in range(HEADS)],
        axis=1)
    o = acc_ref[...] / denom
    oh = (lab == jax.lax.broadcasted_iota(jnp.int32, (1, TASKS), 1)
          ).astype(jnp.float32)
    proj_ref[...] = jnp.zeros((BLK, LAT), jnp.float32)
    for t in range(TASKS):
        @pl.when((t >= tl) & (t <= th))
        def _(t=t):
            proj_ref[...] += oh[:, t:t + 1] * jnp.dot(
                o, wo[t], preferred_element_type=jnp.float32)
    gamma_b = jnp.dot(oh, gamma[...], preferred_element_type=jnp.float32)
    beta_b = jnp.dot(oh, beta[...], preferred_element_type=jnp.float32)
    hr = sb[...] + proj_ref[...]
    mu = jnp.mean(hr, axis=1, keepdims=True)
    var = jnp.mean((hr - mu) ** 2, axis=1, keepdims=True)
    out[...] = (hr - mu) * lax.rsqrt(var + 1e-5) * gamma_b + beta_b


def _pt_attn_layer(s, lab_col, lab_chunks, tfl, tfh, blo, bhi,
                   wq, wk, wv, wo, gamma, beta):
    q, k, v = _pt_qkv(s, lab_col, tfl, tfh, wq, wk, wv)
    return pl.pallas_call(
        _pt_attn_body,
        grid=(NBLK,),
        in_specs=[_SMEM, _SMEM, _SMEM, _SMEM,
                  _rows((BLK, LAT)), _rows((BLK, LAT)), _rows((BLK, 1)),
                  _full((NCH, 1, CH)), _full((NCH, LAT, CH)),
                  _full((HEADS, N, DHE)),
                  _full((TASKS, LAT, LAT)), _full((TASKS, LAT)),
                  _full((TASKS, LAT))],
        out_specs=_rows((BLK, LAT)),
        out_shape=jax.ShapeDtypeStruct((N, LAT), jnp.float32),
        scratch_shapes=[pltpu.VMEM((BLK, LAT), jnp.float32),
                        pltpu.VMEM((BLK, HEADS), jnp.float32),
                        pltpu.VMEM((BLK, HEADS), jnp.float32),
                        pltpu.VMEM((BLK, LAT), jnp.float32)],
        compiler_params=_PARALLEL,
    )(tfl, tfh, blo, bhi, s, q, lab_col, lab_chunks, k, v, wo, gamma, beta)


def _g_qkv_body(sb, wq, wk, wv, qo, kto, vo):
    s = sb[...]
    qo[...] = jnp.dot(s, wq[...], preferred_element_type=jnp.float32)
    kto[...] = jnp.dot(s, wk[...], preferred_element_type=jnp.float32).T
    v = jnp.dot(s, wv[...], preferred_element_type=jnp.float32)
    ones = jnp.ones((BLK, 8), jnp.float32)
    vo[0] = jnp.concatenate([v[:, :DH], ones], axis=1)
    vo[1] = jnp.concatenate([v[:, DH:], ones], axis=1)


def _g_attn_body(sb, qb, ktf, vf, wo, gamma, beta, out):
    q = qb[...]
    kt = ktf[...]
    outs = []
    for h in range(HEADS):
        qh = q[:, h * DH:(h + 1) * DH] * (0.125 * LOG2E)
        kth = kt[h * DH:(h + 1) * DH, :]
        logits = lax.dot_general(
            qh, kth, (((1,), (0,)), ((), ())),
            preferred_element_type=jnp.float32)
        m = jnp.max(logits, axis=1, keepdims=True)
        e = jnp.exp2(logits - m)
        avd = jnp.dot(e, vf[h], preferred_element_type=jnp.float32)
        outs.append(avd[:, :DH] / avd[:, DH:DH + 1])
    o = jnp.concatenate(outs, axis=1)
    proj = jnp.dot(o, wo[...], preferred_element_type=jnp.float32)
    hr = sb[...] + proj
    mu = jnp.mean(hr, axis=1, keepdims=True)
    var = jnp.mean((hr - mu) ** 2, axis=1, keepdims=True)
    out[...] = (hr - mu) * lax.rsqrt(var + 1e-5) * gamma[...] + beta[...]


def _g_attn_layer(s, wq, wk, wv, wo, gamma, beta):
    out = jax.ShapeDtypeStruct((N, LAT), jnp.float32)
    out_kt = jax.ShapeDtypeStruct((LAT, N), jnp.float32)
    out_v = jax.ShapeDtypeStruct((HEADS, N, DHE), jnp.float32)
    q, kt, v = pl.pallas_call(
        _g_qkv_body,
        grid=(NBLK,),
        in_specs=[_rows((BLK, LAT))] + [_full((LAT, LAT))] * 3,
        out_specs=[_rows((BLK, LAT)),
                   pl.BlockSpec((LAT, BLK), lambda i: (0, i)),
                   pl.BlockSpec((HEADS, BLK, DHE), lambda i: (0, i, 0))],
        out_shape=[out, out_kt, out_v],
        compiler_params=_PARALLEL,
    )(s, wq, wk, wv)
    return pl.pallas_call(
        _g_attn_body,
        grid=(N // BLKG,),
        in_specs=[_rows((BLKG, LAT)), _rows((BLKG, LAT)),
                  _full((LAT, N)), _full((HEADS, N, DHE)),
                  _full((LAT, LAT)), _full((1, LAT)), _full((1, LAT))],
        out_specs=_rows((BLKG, LAT)),
        out_shape=out,
        compiler_params=_PARALLEL,
    )(s, q, kt, v, wo, gamma, beta)


def _head_body(tf, w0, b0, w1, b1, wmu, bmu, wsig, bsig, muo, sigo):
    m = jnp.mean(tf[...], axis=0, keepdims=True)
    h = jnp.maximum(jnp.dot(m, w0[...], preferred_element_type=jnp.float32)
                    + b0[...], 0.0)
    h = jnp.dot(h, w1[...], preferred_element_type=jnp.float32) + b1[...]
    muo[...] = jnp.dot(h, wmu[...], preferred_element_type=jnp.float32) + bmu[...]
    z = jnp.dot(h, wsig[...], preferred_element_type=jnp.float32) + bsig[...]
    sigo[...] = 0.1 + 0.9 * jax.nn.sigmoid(z)


def _head(t, w0, b0, w1, b1, wmu, bmu, wsig, bsig):
    out = jax.ShapeDtypeStruct((1, LAT), jnp.float32)
    return pl.pallas_call(
        _head_body,
        grid=(1,),
        in_specs=[_full((N, LAT))] + [_full((LAT, LAT)), _full((1, LAT))] * 4,
        out_specs=[_full((1, LAT))] * 2,
        out_shape=[out, out],
    )(t, w0, b0, w1, b1, wmu, bmu, wsig, bsig)


def kernel(x, y, task_labels, set_W0, set_b0, set_W1, set_b1,
           pt_Wq, pt_Wk, pt_Wv, pt_Wo, pt_gamma, pt_beta,
           g_Wq, g_Wk, g_Wv, g_Wo, g_gamma, g_beta,
           am_W0, am_b0, am_W1, am_b1, am_Wmu, am_bmu, am_Wsig, am_bsig):
    r = lambda b: b.reshape(1, LAT)

    lab = task_labels.astype(jnp.int32)
    tids = jnp.arange(TASKS, dtype=jnp.int32)
    oh = (lab[:, None] == tids[None, :]).astype(jnp.int32)
    counts = oh.sum(0)
    ends = jnp.cumsum(counts)
    starts = ends - counts
    cc = jnp.cumsum(oh, axis=0)
    rank = ((oh * starts[None, :]).sum(1) + (oh * cc).sum(1) - 1
            ).astype(jnp.int32)
    pos = jnp.arange(N, dtype=jnp.int32)
    lab_sorted = (pos[:, None] >= ends[None, :]).sum(1).astype(jnp.int32)
    lab_col = lab_sorted.reshape(N, 1)
    lab_chunks = lab_sorted.reshape(NCH, 1, CH)
    tfl = lab_sorted[::BLK]
    tfh = lab_sorted[BLK - 1::BLK]
    ohl = (tfl[:, None] == tids[None, :]).astype(jnp.int32)
    ohh = (tfh[:, None] == tids[None, :]).astype(jnp.int32)
    kstart = (ohl * starts[None, :]).sum(1)
    kend = (ohh * ends[None, :]).sum(1)
    blo = (kstart // CH).astype(jnp.int32)
    bhi = ((kend - 1) // CH).astype(jnp.int32)
    idx2d = rank.reshape(_NW, BPW)

    s = _set_mlp(x, y, set_W0[:x.shape[1]], set_W0[x.shape[1]:],
                 r(set_b0), set_W1, r(set_b1))

    sl = _sc_permute(s, idx2d, "scatter")
    for l in range(pt_Wq.shape[1]):
        sl = _pt_attn_layer(sl, lab_col, lab_chunks, tfl, tfh, blo, bhi,
                            pt_Wq[:, l], pt_Wk[:, l], pt_Wv[:, l],
                            pt_Wo[:, l], pt_gamma[:, l], pt_beta[:, l])

    t = sl
    for l in range(g_Wq.shape[0]):
        t = _g_attn_layer(t, g_Wq[l], g_Wk[l], g_Wv[l], g_Wo[l],
                          r(g_gamma[l]), r(g_beta[l]))

    mu, sig = _head(t, am_W0, r(am_b0), am_W1, r(am_b1),
                    am_Wmu, r(am_bmu), am_Wsig, r(am_bsig))
    s_local = _sc_permute(sl, idx2d, "gather")
    temp = _sc_permute(t, idx2d, "gather")
    return mu.reshape(LAT), sig.reshape(LAT), s_local, temp


# speedup vs baseline: 1.1788x; 1.0538x over previous
import functools

import jax
import jax.numpy as jnp
from jax import lax
from jax.experimental import pallas as pl
from jax.experimental.pallas import tpu as pltpu
from jax.experimental.pallas import tpu_sc as plsc

N = 4096
LAT = 128
HEADS = 2
DH = LAT // HEADS
TASKS = 8
BLK = 512
NBLK = N // BLK
BLKG = 1024
DHE = DH + 8
LOG2E = 1.4426950408889634
CH = 1024
NCH = N // CH
NEG = -1e30

_SC_CORES = 2
_SC_SUBCORES = 16
_NW = _SC_CORES * _SC_SUBCORES
BPW = N // _NW


def _full(shape):
    return pl.BlockSpec(shape, lambda i: tuple(0 for _ in shape))


def _rows(shape):
    return pl.BlockSpec(shape, lambda i: (i,) + tuple(0 for _ in shape[1:]))


_SMEM = pl.BlockSpec(memory_space=pltpu.SMEM)
_PARALLEL = pltpu.CompilerParams(dimension_semantics=("parallel",))


def _sc_permute(src, idx2d, direction):
    mesh = plsc.VectorSubcoreMesh(core_axis_name="c", subcore_axis_name="s",
                                  num_cores=_SC_CORES,
                                  num_subcores=_SC_SUBCORES)

    @functools.partial(
        pl.kernel, mesh=mesh,
        out_type=jax.ShapeDtypeStruct((N, LAT), jnp.float32),
        scratch_types=[pltpu.VMEM((BPW,), jnp.int32),
                       pltpu.VMEM((BPW, LAT), jnp.float32),
                       pltpu.SemaphoreType.DMA],
    )
    def k(src_hbm, idx_hbm, out_hbm, idx_v, rows_v, sem):
        wid = lax.axis_index("s") * _SC_CORES + lax.axis_index("c")
        base = wid * BPW
        pltpu.sync_copy(idx_hbm.at[wid], idx_v)
        if direction == "scatter":
            pltpu.sync_copy(src_hbm.at[pl.ds(base, BPW)], rows_v)
            pltpu.async_copy(rows_v, out_hbm.at[idx_v], sem).wait()
        else:
            pltpu.async_copy(src_hbm.at[idx_v], rows_v, sem).wait()
            pltpu.sync_copy(rows_v, out_hbm.at[pl.ds(base, BPW)])

    return k(src, idx2d)


def _set_mlp_body(xb, yb, w0x, w0y, b0, w1, b1, out):
    h = (jnp.dot(xb[...], w0x[...], preferred_element_type=jnp.float32)
         + jnp.dot(yb[...], w0y[...], preferred_element_type=jnp.float32)
         + b0[...])
    h = jnp.maximum(h, 0.0)
    out[...] = jnp.dot(h, w1[...], preferred_element_type=jnp.float32) + b1[...]


def _set_mlp(x, y, w0x, w0y, b0, w1, b1):
    return pl.pallas_call(
        _set_mlp_body,
        grid=(NBLK,),
        in_specs=[_rows((BLK, x.shape[1])), _rows((BLK, y.shape[1])),
                  _full(w0x.shape), _full(w0y.shape), _full((1, LAT)),
                  _full((LAT, LAT)), _full((1, LAT))],
        out_specs=_rows((BLK, LAT)),
        out_shape=jax.ShapeDtypeStruct((N, LAT), jnp.float32),
        compiler_params=_PARALLEL,
    )(x, y, w0x, w0y, b0, w1, b1)


def _pt_qkv_body(tfl, tfh, sb, labb, wq, wk, wv, qo, kto, vo, kacc, vacc):
    b = pl.program_id(0)
    tl = tfl[b]
    th = tfh[b]
    s = sb[...]
    lab = labb[...]
    oh = (lab == jax.lax.broadcasted_iota(jnp.int32, (1, TASKS), 1)
          ).astype(jnp.float32)
    qo[...] = jnp.zeros((BLK, LAT), jnp.float32)
    kacc[...] = jnp.zeros((BLK, LAT), jnp.float32)
    vacc[...] = jnp.zeros((BLK, LAT), jnp.float32)
    for t in range(TASKS):
        @pl.when((t >= tl) & (t <= th))
        def _(t=t):
            m = oh[:, t:t + 1]
            qo[...] += m * jnp.dot(s, wq[t], preferred_element_type=jnp.float32)
            kacc[...] += m * jnp.dot(s, wk[t], preferred_element_type=jnp.float32)
            vacc[...] += m * jnp.dot(s, wv[t], preferred_element_type=jnp.float32)
    kto[0] = kacc[...].T
    ones = jnp.ones((BLK, 8), jnp.float32)
    v = vacc[...]
    vo[0] = jnp.concatenate([v[:, :DH], ones], axis=1)
    vo[1] = jnp.concatenate([v[:, DH:], ones], axis=1)


def _pt_qkv(s, lab_col, tfl, tfh, wq, wk, wv):
    out = jax.ShapeDtypeStruct((N, LAT), jnp.float32)
    out_kt = jax.ShapeDtypeStruct((NCH, LAT, CH), jnp.float32)
    out_v = jax.ShapeDtypeStruct((HEADS, N, DHE), jnp.float32)
    cpb = CH // BLK
    return pl.pallas_call(
        _pt_qkv_body,
        grid=(NBLK,),
        in_specs=[_SMEM, _SMEM, _rows((BLK, LAT)), _rows((BLK, 1)),
                  _full((TASKS, LAT, LAT)), _full((TASKS, LAT, LAT)),
                  _full((TASKS, LAT, LAT))],
        out_specs=[_rows((BLK, LAT)),
                   pl.BlockSpec((1, LAT, BLK),
                                lambda i: (i // cpb, 0, i % cpb)),
                   pl.BlockSpec((HEADS, BLK, DHE), lambda i: (0, i, 0))],
        out_shape=[out, out_kt, out_v],
        scratch_shapes=[pltpu.VMEM((BLK, LAT), jnp.float32),
                        pltpu.VMEM((BLK, LAT), jnp.float32)],
        compiler_params=_PARALLEL,
    )(tfl, tfh, s, lab_col, wq, wk, wv)


def _pt_attn_body(tfl, tfh, blo, bhi, sb, qb, labb, labch, kf, vf,
                  wo, gamma, beta, out, acc_ref, m_ref, l_ref, proj_ref):
    b = pl.program_id(0)
    lo = blo[b]
    hi = bhi[b]
    tl = tfl[b]
    th = tfh[b]
    q = qb[...]
    lab = labb[...]
    j0 = b // (CH // BLK)
    madd0 = jnp.where(lab == labch[j0], 0.0, NEG)
    kt0 = kf[j0]
    for h in range(HEADS):
        qh = q[:, h * DH:(h + 1) * DH] * (0.125 * LOG2E)
        logits = lax.dot_general(
            qh, kt0[h * DH:(h + 1) * DH, :], (((1,), (0,)), ((), ())),
            preferred_element_type=jnp.float32) + madd0
        m = jnp.max(logits, 1, keepdims=True)
        p = jnp.exp2(logits - m)
        m_ref[:, h:h + 1] = m
        avd = jnp.dot(p, vf[h, pl.ds(j0 * CH, CH), :],
                      preferred_element_type=jnp.float32)
        l_ref[:, h:h + 1] = avd[:, DH:DH + 1]
        acc_ref[:, h * DH:(h + 1) * DH] = avd[:, :DH]
    for j in range(NCH):
        @pl.when((j >= lo) & (j <= hi) & (j != j0))
        def _(j=j):
            madd = jnp.where(lab == labch[j], 0.0, NEG)
            for h in range(HEADS):
                qh = q[:, h * DH:(h + 1) * DH] * (0.125 * LOG2E)
                kth = kf[j, h * DH:(h + 1) * DH, :]
                logits = lax.dot_general(
                    qh, kth, (((1,), (0,)), ((), ())),
                    preferred_element_type=jnp.float32) + madd
                mprev = m_ref[:, h:h + 1]
                mnew = jnp.maximum(mprev, jnp.max(logits, 1, keepdims=True))
                p = jnp.exp2(logits - mnew)
                scale = jnp.exp2(mprev - mnew)
                avd = jnp.dot(p, vf[h, j * CH:(j + 1) * CH, :],
                              preferred_element_type=jnp.float32)
                l_ref[:, h:h + 1] = (l_ref[:, h:h + 1] * scale
                                     + avd[:, DH:DH + 1])
                acc_ref[:, h * DH:(h + 1) * DH] = (
                    acc_ref[:, h * DH:(h + 1) * DH] * scale + avd[:, :DH])
                m_ref[:, h:h + 1] = mnew
    denom = jnp.concatenate(
        [jnp.broadcast_to(l_ref[:, h:h + 1], (BLK, DH)) for h in range(HEADS)],
        axis=1)
    o = acc_ref[...] / denom
    oh = (lab == jax.lax.broadcasted_iota(jnp.int32, (1, TASKS), 1)
          ).astype(jnp.float32)
    proj_ref[...] = jnp.zeros((BLK, LAT), jnp.float32)
    for t in range(TASKS):
        @pl.when((t >= tl) & (t <= th))
        def _(t=t):
            proj_ref[...] += oh[:, t:t + 1] * jnp.dot(
                o, wo[t], preferred_element_type=jnp.float32)
    gamma_b = jnp.dot(oh, gamma[...], preferred_element_type=jnp.float32)
    beta_b = jnp.dot(oh, beta[...], preferred_element_type=jnp.float32)
    hr = sb[...] + proj_ref[...]
    mu = jnp.mean(hr, axis=1, keepdims=True)
    var = jnp.mean((hr - mu) ** 2, axis=1, keepdims=True)
    out[...] = (hr - mu) * lax.rsqrt(var + 1e-5) * gamma_b + beta_b


def _pt_attn_body_fused(tfl, tfh, blo, bhi, sb, qb, labb, labch, kf, vf,
                        wo, gamma, beta, gwq, gwk, gwv,
                        out, qo2, kto2, vo2,
                        acc_ref, m_ref, l_ref, proj_ref):
    _pt_attn_body(tfl, tfh, blo, bhi, sb, qb, labb, labch, kf, vf,
                  wo, gamma, beta, out, acc_ref, m_ref, l_ref, proj_ref)
    res = out[...]
    qo2[...] = jnp.dot(res, gwq[...], preferred_element_type=jnp.float32)
    kto2[...] = jnp.dot(res, gwk[...], preferred_element_type=jnp.float32).T
    v2 = jnp.dot(res, gwv[...], preferred_element_type=jnp.float32)
    ones = jnp.ones((BLK, 8), jnp.float32)
    vo2[0] = jnp.concatenate([v2[:, :DH], ones], axis=1)
    vo2[1] = jnp.concatenate([v2[:, DH:], ones], axis=1)


def _pt_attn_layer(s, lab_col, lab_chunks, tfl, tfh, blo, bhi,
                   wq, wk, wv, wo, gamma, beta, fuse_w=None):
    q, k, v = _pt_qkv(s, lab_col, tfl, tfh, wq, wk, wv)
    out = jax.ShapeDtypeStruct((N, LAT), jnp.float32)
    in_specs = [_SMEM, _SMEM, _SMEM, _SMEM,
                _rows((BLK, LAT)), _rows((BLK, LAT)), _rows((BLK, 1)),
                _full((NCH, 1, CH)), _full((NCH, LAT, CH)),
                _full((HEADS, N, DHE)),
                _full((TASKS, LAT, LAT)), _full((TASKS, LAT)),
                _full((TASKS, LAT))]
    scratch = [pltpu.VMEM((BLK, LAT), jnp.float32),
               pltpu.VMEM((BLK, HEADS), jnp.float32),
               pltpu.VMEM((BLK, HEADS), jnp.float32),
               pltpu.VMEM((BLK, LAT), jnp.float32)]
    args = (tfl, tfh, blo, bhi, s, q, lab_col, lab_chunks, k, v,
            wo, gamma, beta)
    if fuse_w is None:
        return pl.pallas_call(
            _pt_attn_body,
            grid=(NBLK,),
            in_specs=in_specs,
            out_specs=_rows((BLK, LAT)),
            out_shape=out,
            scratch_shapes=scratch,
            compiler_params=_PARALLEL,
        )(*args)
    return pl.pallas_call(
        _pt_attn_body_fused,
        grid=(NBLK,),
        in_specs=in_specs + [_full((LAT, LAT))] * 3,
        out_specs=[_rows((BLK, LAT)), _rows((BLK, LAT)),
                   pl.BlockSpec((LAT, BLK), lambda i: (0, i)),
                   pl.BlockSpec((HEADS, BLK, DHE), lambda i: (0, i, 0))],
        out_shape=[out, out,
                   jax.ShapeDtypeStruct((LAT, N), jnp.float32),
                   jax.ShapeDtypeStruct((HEADS, N, DHE), jnp.float32)],
        scratch_shapes=scratch,
        compiler_params=_PARALLEL,
    )(*(args + tuple(fuse_w)))


def _g_qkv_body(sb, wq, wk, wv, qo, kto, vo):
    s = sb[...]
    qo[...] = jnp.dot(s, wq[...], preferred_element_type=jnp.float32)
    kto[...] = jnp.dot(s, wk[...], preferred_element_type=jnp.float32).T
    v = jnp.dot(s, wv[...], preferred_element_type=jnp.float32)
    ones = jnp.ones((BLK, 8), jnp.float32)
    vo[0] = jnp.concatenate([v[:, :DH], ones], axis=1)
    vo[1] = jnp.concatenate([v[:, DH:], ones], axis=1)


def _g_attn_body(sb, qb, ktf, vf, wo, gamma, beta, out):
    q = qb[...]
    kt = ktf[...]
    outs = []
    for h in range(HEADS):
        qh = q[:, h * DH:(h + 1) * DH] * (0.125 * LOG2E)
        kth = kt[h * DH:(h + 1) * DH, :]
        logits = lax.dot_general(
            qh, kth, (((1,), (0,)), ((), ())),
            preferred_element_type=jnp.float32)
        m = jnp.max(logits, axis=1, keepdims=True)
        e = jnp.exp2(logits - m)
        avd = jnp.dot(e, vf[h], preferred_element_type=jnp.float32)
        outs.append(avd[:, :DH] / avd[:, DH:DH + 1])
    o = jnp.concatenate(outs, axis=1)
    proj = jnp.dot(o, wo[...], preferred_element_type=jnp.float32)
    hr = sb[...] + proj
    mu = jnp.mean(hr, axis=1, keepdims=True)
    var = jnp.mean((hr - mu) ** 2, axis=1, keepdims=True)
    out[...] = (hr - mu) * lax.rsqrt(var + 1e-5) * gamma[...] + beta[...]


def _g_attn_body_fused(sb, qb, ktf, vf, wo, gamma, beta, gwq, gwk, gwv,
                       out, qo2, kto2, vo2):
    _g_attn_body(sb, qb, ktf, vf, wo, gamma, beta, out)
    res = out[...]
    qo2[...] = jnp.dot(res, gwq[...], preferred_element_type=jnp.float32)
    kto2[...] = jnp.dot(res, gwk[...], preferred_element_type=jnp.float32).T
    v2 = jnp.dot(res, gwv[...], preferred_element_type=jnp.float32)
    ones = jnp.ones((BLKG, 8), jnp.float32)
    vo2[0] = jnp.concatenate([v2[:, :DH], ones], axis=1)
    vo2[1] = jnp.concatenate([v2[:, DH:], ones], axis=1)


def _g_attn_call(s, q, kt, v, wo, gamma, beta, fuse_w=None):
    out = jax.ShapeDtypeStruct((N, LAT), jnp.float32)
    in_specs = [_rows((BLKG, LAT)), _rows((BLKG, LAT)),
                _full((LAT, N)), _full((HEADS, N, DHE)),
                _full((LAT, LAT)), _full((1, LAT)), _full((1, LAT))]
    if fuse_w is None:
        return pl.pallas_call(
            _g_attn_body,
            grid=(N // BLKG,),
            in_specs=in_specs,
            out_specs=_rows((BLKG, LAT)),
            out_shape=out,
            compiler_params=_PARALLEL,
        )(s, q, kt, v, wo, gamma, beta)
    return pl.pallas_call(
        _g_attn_body_fused,
        grid=(N // BLKG,),
        in_specs=in_specs + [_full((LAT, LAT))] * 3,
        out_specs=[_rows((BLKG, LAT)), _rows((BLKG, LAT)),
                   pl.BlockSpec((LAT, BLKG), lambda i: (0, i)),
                   pl.BlockSpec((HEADS, BLKG, DHE), lambda i: (0, i, 0))],
        out_shape=[out, out,
                   jax.ShapeDtypeStruct((LAT, N), jnp.float32),
                   jax.ShapeDtypeStruct((HEADS, N, DHE), jnp.float32)],
        compiler_params=_PARALLEL,
    )(s, q, kt, v, wo, gamma, beta, *fuse_w)


def _head_body(tf, w0, b0, w1, b1, wmu, bmu, wsig, bsig, muo, sigo):
    m = jnp.mean(tf[...], axis=0, keepdims=True)
    h = jnp.maximum(jnp.dot(m, w0[...], preferred_element_type=jnp.float32)
                    + b0[...], 0.0)
    h = jnp.dot(h, w1[...], preferred_element_type=jnp.float32) + b1[...]
    muo[...] = jnp.dot(h, wmu[...], preferred_element_type=jnp.float32) + bmu[...]
    z = jnp.dot(h, wsig[...], preferred_element_type=jnp.float32) + bsig[...]
    sigo[...] = 0.1 + 0.9 * jax.nn.sigmoid(z)


def _head(t, w0, b0, w1, b1, wmu, bmu, wsig, bsig):
    out = jax.ShapeDtypeStruct((1, LAT), jnp.float32)
    return pl.pallas_call(
        _head_body,
        grid=(1,),
        in_specs=[_full((N, LAT))] + [_full((LAT, LAT)), _full((1, LAT))] * 4,
        out_specs=[_full((1, LAT))] * 2,
        out_shape=[out, out],
    )(t, w0, b0, w1, b1, wmu, bmu, wsig, bsig)


def kernel(x, y, task_labels, set_W0, set_b0, set_W1, set_b1,
           pt_Wq, pt_Wk, pt_Wv, pt_Wo, pt_gamma, pt_beta,
           g_Wq, g_Wk, g_Wv, g_Wo, g_gamma, g_beta,
           am_W0, am_b0, am_W1, am_b1, am_Wmu, am_bmu, am_Wsig, am_bsig):
    r = lambda b: b.reshape(1, LAT)

    lab = task_labels.astype(jnp.int32)
    tids = jnp.arange(TASKS, dtype=jnp.int32)
    oh = (lab[:, None] == tids[None, :]).astype(jnp.int32)
    counts = oh.sum(0)
    ends = jnp.cumsum(counts)
    starts = ends - counts
    cc = jnp.cumsum(oh, axis=0)
    rank = ((oh * starts[None, :]).sum(1) + (oh * cc).sum(1) - 1
            ).astype(jnp.int32)
    pos = jnp.arange(N, dtype=jnp.int32)
    lab_sorted = (pos[:, None] >= ends[None, :]).sum(1).astype(jnp.int32)
    lab_col = lab_sorted.reshape(N, 1)
    lab_chunks = lab_sorted.reshape(NCH, 1, CH)
    tfl = lab_sorted[::BLK]
    tfh = lab_sorted[BLK - 1::BLK]
    ohl = (tfl[:, None] == tids[None, :]).astype(jnp.int32)
    ohh = (tfh[:, None] == tids[None, :]).astype(jnp.int32)
    kstart = (ohl * starts[None, :]).sum(1)
    kend = (ohh * ends[None, :]).sum(1)
    blo = (kstart // CH).astype(jnp.int32)
    bhi = ((kend - 1) // CH).astype(jnp.int32)
    idx2d = rank.reshape(_NW, BPW)

    s = _set_mlp(x, y, set_W0[:x.shape[1]], set_W0[x.shape[1]:],
                 r(set_b0), set_W1, r(set_b1))

    sl = _sc_permute(s, idx2d, "scatter")
    sl = _pt_attn_layer(sl, lab_col, lab_chunks, tfl, tfh, blo, bhi,
                        pt_Wq[:, 0], pt_Wk[:, 0], pt_Wv[:, 0],
                        pt_Wo[:, 0], pt_gamma[:, 0], pt_beta[:, 0])
    sl, gq, gkt, gv = _pt_attn_layer(
        sl, lab_col, lab_chunks, tfl, tfh, blo, bhi,
        pt_Wq[:, 1], pt_Wk[:, 1], pt_Wv[:, 1],
        pt_Wo[:, 1], pt_gamma[:, 1], pt_beta[:, 1],
        fuse_w=(g_Wq[0], g_Wk[0], g_Wv[0]))

    t, gq2, gkt2, gv2 = _g_attn_call(
        sl, gq, gkt, gv, g_Wo[0], r(g_gamma[0]), r(g_beta[0]),
        fuse_w=(g_Wq[1], g_Wk[1], g_Wv[1]))
    t = _g_attn_call(t, gq2, gkt2, gv2, g_Wo[1], r(g_gamma[1]), r(g_beta[1]))

    mu, sig = _head(t, am_W0, r(am_b0), am_W1, r(am_b1),
                    am_Wmu, r(am_bmu), am_Wsig, r(am_bsig))
    s_local = _sc_permute(sl, idx2d, "gather")
    temp = _sc_permute(t, idx2d, "gather")
    return mu.reshape(LAT), sig.reshape(LAT), s_local, temp


# speedup vs baseline: 1.1899x; 1.0094x over previous
import functools

import jax
import jax.numpy as jnp
from jax import lax
from jax.experimental import pallas as pl
from jax.experimental.pallas import tpu as pltpu
from jax.experimental.pallas import tpu_sc as plsc

N = 4096
LAT = 128
HEADS = 2
DH = LAT // HEADS
TASKS = 8
BLK = 512
NBLK = N // BLK
BLKG = 1024
DHE = DH + 8
LOG2E = 1.4426950408889634
CH = 1024
NCH = N // CH
NEG = -1e30

_SC_CORES = 2
_SC_SUBCORES = 16
_NW = _SC_CORES * _SC_SUBCORES
BPW = N // _NW


def _full(shape):
    return pl.BlockSpec(shape, lambda i: tuple(0 for _ in shape))


def _rows(shape):
    return pl.BlockSpec(shape, lambda i: (i,) + tuple(0 for _ in shape[1:]))


_SMEM = pl.BlockSpec(memory_space=pltpu.SMEM)
_PARALLEL = pltpu.CompilerParams(dimension_semantics=("parallel",))


def _sc_permute(src, idx2d, direction):
    mesh = plsc.VectorSubcoreMesh(core_axis_name="c", subcore_axis_name="s",
                                  num_cores=_SC_CORES,
                                  num_subcores=_SC_SUBCORES)

    @functools.partial(
        pl.kernel, mesh=mesh,
        out_type=jax.ShapeDtypeStruct((N, LAT), jnp.float32),
        scratch_types=[pltpu.VMEM((BPW,), jnp.int32),
                       pltpu.VMEM((BPW, LAT), jnp.float32),
                       pltpu.SemaphoreType.DMA],
    )
    def k(src_hbm, idx_hbm, out_hbm, idx_v, rows_v, sem):
        wid = lax.axis_index("s") * _SC_CORES + lax.axis_index("c")
        base = wid * BPW
        pltpu.sync_copy(idx_hbm.at[wid], idx_v)
        if direction == "scatter":
            pltpu.sync_copy(src_hbm.at[pl.ds(base, BPW)], rows_v)
            pltpu.async_copy(rows_v, out_hbm.at[idx_v], sem).wait()
        else:
            pltpu.async_copy(src_hbm.at[idx_v], rows_v, sem).wait()
            pltpu.sync_copy(rows_v, out_hbm.at[pl.ds(base, BPW)])

    return k(src, idx2d)


def _set_mlp_body(xb, yb, w0x, w0y, b0, w1, b1, out):
    h = (jnp.dot(xb[...], w0x[...], preferred_element_type=jnp.float32)
         + jnp.dot(yb[...], w0y[...], preferred_element_type=jnp.float32)
         + b0[...])
    h = jnp.maximum(h, 0.0)
    out[...] = jnp.dot(h, w1[...], preferred_element_type=jnp.float32) + b1[...]


def _set_mlp(x, y, w0x, w0y, b0, w1, b1):
    return pl.pallas_call(
        _set_mlp_body,
        grid=(NBLK,),
        in_specs=[_rows((BLK, x.shape[1])), _rows((BLK, y.shape[1])),
                  _full(w0x.shape), _full(w0y.shape), _full((1, LAT)),
                  _full((LAT, LAT)), _full((1, LAT))],
        out_specs=_rows((BLK, LAT)),
        out_shape=jax.ShapeDtypeStruct((N, LAT), jnp.float32),
        compiler_params=_PARALLEL,
    )(x, y, w0x, w0y, b0, w1, b1)


def _pt_qkv_body(tfl, tfh, sb, labb, wq, wk, wv, qo, kto, vo, kacc, vacc):
    b = pl.program_id(0)
    tl = tfl[b]
    th = tfh[b]
    s = sb[...]
    lab = labb[...]
    oh = (lab == jax.lax.broadcasted_iota(jnp.int32, (1, TASKS), 1)
          ).astype(jnp.float32)
    qo[...] = jnp.zeros((BLK, LAT), jnp.float32)
    kacc[...] = jnp.zeros((BLK, LAT), jnp.float32)
    vacc[...] = jnp.zeros((BLK, LAT), jnp.float32)
    for t in range(TASKS):
        @pl.when((t >= tl) & (t <= th))
        def _(t=t):
            m = oh[:, t:t + 1]
            qo[...] += m * jnp.dot(s, wq[t], preferred_element_type=jnp.float32)
            kacc[...] += m * jnp.dot(s, wk[t], preferred_element_type=jnp.float32)
            vacc[...] += m * jnp.dot(s, wv[t], preferred_element_type=jnp.float32)
    kto[0] = kacc[...].T
    ones = jnp.ones((BLK, 8), jnp.float32)
    v = vacc[...]
    vo[0] = jnp.concatenate([v[:, :DH], ones], axis=1)
    vo[1] = jnp.concatenate([v[:, DH:], ones], axis=1)


def _pt_qkv(s, lab_col, tfl, tfh, wq, wk, wv):
    out = jax.ShapeDtypeStruct((N, LAT), jnp.float32)
    out_kt = jax.ShapeDtypeStruct((NCH, LAT, CH), jnp.float32)
    out_v = jax.ShapeDtypeStruct((HEADS, N, DHE), jnp.float32)
    cpb = CH // BLK
    return pl.pallas_call(
        _pt_qkv_body,
        grid=(NBLK,),
        in_specs=[_SMEM, _SMEM, _rows((BLK, LAT)), _rows((BLK, 1)),
                  _full((TASKS, LAT, LAT)), _full((TASKS, LAT, LAT)),
                  _full((TASKS, LAT, LAT))],
        out_specs=[_rows((BLK, LAT)),
                   pl.BlockSpec((1, LAT, BLK),
                                lambda i: (i // cpb, 0, i % cpb)),
                   pl.BlockSpec((HEADS, BLK, DHE), lambda i: (0, i, 0))],
        out_shape=[out, out_kt, out_v],
        scratch_shapes=[pltpu.VMEM((BLK, LAT), jnp.float32),
                        pltpu.VMEM((BLK, LAT), jnp.float32)],
        compiler_params=_PARALLEL,
    )(tfl, tfh, s, lab_col, wq, wk, wv)


def _pt_attn_body(tfl, tfh, blo, bhi, sb, qb, labb, labch, kf, vf,
                  wo, gamma, beta, out, acc_ref, m_ref, l_ref, proj_ref):
    b = pl.program_id(0)
    lo = blo[b]
    hi = bhi[b]
    tl = tfl[b]
    th = tfh[b]
    q = qb[...]
    lab = labb[...]
    j0 = b // (CH // BLK)
    madd0 = jnp.where(lab == labch[j0], 0.0, NEG)
    kt0 = kf[j0]
    for h in range(HEADS):
        qh = q[:, h * DH:(h + 1) * DH] * (0.125 * LOG2E)
        logits = lax.dot_general(
            qh, kt0[h * DH:(h + 1) * DH, :], (((1,), (0,)), ((), ())),
            preferred_element_type=jnp.float32) + madd0
        m = jnp.max(logits, 1, keepdims=True)
        p = jnp.exp2(logits - m)
        m_ref[:, h:h + 1] = m
        avd = jnp.dot(p, vf[h, pl.ds(j0 * CH, CH), :],
                      preferred_element_type=jnp.float32)
        l_ref[:, h:h + 1] = avd[:, DH:DH + 1]
        acc_ref[:, h * DH:(h + 1) * DH] = avd[:, :DH]
    for j in range(NCH):
        @pl.when((j >= lo) & (j <= hi) & (j != j0))
        def _(j=j):
            madd = jnp.where(lab == labch[j], 0.0, NEG)
            for h in range(HEADS):
                qh = q[:, h * DH:(h + 1) * DH] * (0.125 * LOG2E)
                kth = kf[j, h * DH:(h + 1) * DH, :]
                logits = lax.dot_general(
                    qh, kth, (((1,), (0,)), ((), ())),
                    preferred_element_type=jnp.float32) + madd
                mprev = m_ref[:, h:h + 1]
                mnew = jnp.maximum(mprev, jnp.max(logits, 1, keepdims=True))
                p = jnp.exp2(logits - mnew)
                scale = jnp.exp2(mprev - mnew)
                avd = jnp.dot(p, vf[h, j * CH:(j + 1) * CH, :],
                              preferred_element_type=jnp.float32)
                l_ref[:, h:h + 1] = (l_ref[:, h:h + 1] * scale
                                     + avd[:, DH:DH + 1])
                acc_ref[:, h * DH:(h + 1) * DH] = (
                    acc_ref[:, h * DH:(h + 1) * DH] * scale + avd[:, :DH])
                m_ref[:, h:h + 1] = mnew
    denom = jnp.concatenate(
        [jnp.broadcast_to(l_ref[:, h:h + 1], (BLK, DH)) for h in range(HEADS)],
        axis=1)
    o = acc_ref[...] / denom
    oh = (lab == jax.lax.broadcasted_iota(jnp.int32, (1, TASKS), 1)
          ).astype(jnp.float32)
    proj_ref[...] = jnp.zeros((BLK, LAT), jnp.float32)
    for t in range(TASKS):
        @pl.when((t >= tl) & (t <= th))
        def _(t=t):
            proj_ref[...] += oh[:, t:t + 1] * jnp.dot(
                o, wo[t], preferred_element_type=jnp.float32)
    gamma_b = jnp.dot(oh, gamma[...], preferred_element_type=jnp.float32)
    beta_b = jnp.dot(oh, beta[...], preferred_element_type=jnp.float32)
    hr = sb[...] + proj_ref[...]
    mu = jnp.mean(hr, axis=1, keepdims=True)
    var = jnp.mean((hr - mu) ** 2, axis=1, keepdims=True)
    out[...] = (hr - mu) * lax.rsqrt(var + 1e-5) * gamma_b + beta_b


def _pt_attn_body_fused(tfl, tfh, blo, bhi, sb, qb, labb, labch, kf, vf,
                        wo, gamma, beta, gwq, gwk, gwv,
                        out, qo2, kto2, vo2,
                        acc_ref, m_ref, l_ref, proj_ref):
    _pt_attn_body(tfl, tfh, blo, bhi, sb, qb, labb, labch, kf, vf,
                  wo, gamma, beta, out, acc_ref, m_ref, l_ref, proj_ref)
    res = out[...]
    qo2[...] = jnp.dot(res, gwq[...], preferred_element_type=jnp.float32)
    kto2[...] = jnp.dot(res, gwk[...], preferred_element_type=jnp.float32).T
    v2 = jnp.dot(res, gwv[...], preferred_element_type=jnp.float32)
    ones = jnp.ones((BLK, 8), jnp.float32)
    vo2[0] = jnp.concatenate([v2[:, :DH], ones], axis=1)
    vo2[1] = jnp.concatenate([v2[:, DH:], ones], axis=1)


def _pt_attn_layer(s, lab_col, lab_chunks, tfl, tfh, blo, bhi,
                   wq, wk, wv, wo, gamma, beta, fuse_w=None):
    q, k, v = _pt_qkv(s, lab_col, tfl, tfh, wq, wk, wv)
    out = jax.ShapeDtypeStruct((N, LAT), jnp.float32)
    in_specs = [_SMEM, _SMEM, _SMEM, _SMEM,
                _rows((BLK, LAT)), _rows((BLK, LAT)), _rows((BLK, 1)),
                _full((NCH, 1, CH)), _full((NCH, LAT, CH)),
                _full((HEADS, N, DHE)),
                _full((TASKS, LAT, LAT)), _full((TASKS, LAT)),
                _full((TASKS, LAT))]
    scratch = [pltpu.VMEM((BLK, LAT), jnp.float32),
               pltpu.VMEM((BLK, HEADS), jnp.float32),
               pltpu.VMEM((BLK, HEADS), jnp.float32),
               pltpu.VMEM((BLK, LAT), jnp.float32)]
    args = (tfl, tfh, blo, bhi, s, q, lab_col, lab_chunks, k, v,
            wo, gamma, beta)
    if fuse_w is None:
        return pl.pallas_call(
            _pt_attn_body,
            grid=(NBLK,),
            in_specs=in_specs,
            out_specs=_rows((BLK, LAT)),
            out_shape=out,
            scratch_shapes=scratch,
            compiler_params=_PARALLEL,
        )(*args)
    return pl.pallas_call(
        _pt_attn_body_fused,
        grid=(NBLK,),
        in_specs=in_specs + [_full((LAT, LAT))] * 3,
        out_specs=[_rows((BLK, LAT)), _rows((BLK, LAT)),
                   pl.BlockSpec((LAT, BLK), lambda i: (0, i)),
                   pl.BlockSpec((HEADS, BLK, DHE), lambda i: (0, i, 0))],
        out_shape=[out, out,
                   jax.ShapeDtypeStruct((LAT, N), jnp.float32),
                   jax.ShapeDtypeStruct((HEADS, N, DHE), jnp.float32)],
        scratch_shapes=scratch,
        compiler_params=_PARALLEL,
    )(*(args + tuple(fuse_w)))


def _g_attn_body(sb, qb, ktf, vf, wo, gamma, beta, out):
    q = qb[...]
    kt = ktf[...]
    outs = []
    for h in range(HEADS):
        qh = q[:, h * DH:(h + 1) * DH] * (0.125 * LOG2E)
        kth = kt[h * DH:(h + 1) * DH, :]
        logits = lax.dot_general(
            qh, kth, (((1,), (0,)), ((), ())),
            preferred_element_type=jnp.float32)
        m = jnp.max(logits, axis=1, keepdims=True)
        e = jnp.exp2(logits - m)
        avd = jnp.dot(e, vf[h], preferred_element_type=jnp.float32)
        outs.append(avd[:, :DH] / avd[:, DH:DH + 1])
    o = jnp.concatenate(outs, axis=1)
    proj = jnp.dot(o, wo[...], preferred_element_type=jnp.float32)
    hr = sb[...] + proj
    mu = jnp.mean(hr, axis=1, keepdims=True)
    var = jnp.mean((hr - mu) ** 2, axis=1, keepdims=True)
    out[...] = (hr - mu) * lax.rsqrt(var + 1e-5) * gamma[...] + beta[...]


def _g_attn_body_fused(sb, qb, ktf, vf, wo, gamma, beta, gwq, gwk, gwv,
                       out, qo2, kto2, vo2):
    _g_attn_body(sb, qb, ktf, vf, wo, gamma, beta, out)
    res = out[...]
    qo2[...] = jnp.dot(res, gwq[...], preferred_element_type=jnp.float32)
    kto2[...] = jnp.dot(res, gwk[...], preferred_element_type=jnp.float32).T
    v2 = jnp.dot(res, gwv[...], preferred_element_type=jnp.float32)
    ones = jnp.ones((BLKG, 8), jnp.float32)
    vo2[0] = jnp.concatenate([v2[:, :DH], ones], axis=1)
    vo2[1] = jnp.concatenate([v2[:, DH:], ones], axis=1)


def _g_attn_call(s, q, kt, v, wo, gamma, beta, fuse_w=None):
    out = jax.ShapeDtypeStruct((N, LAT), jnp.float32)
    in_specs = [_rows((BLKG, LAT)), _rows((BLKG, LAT)),
                _full((LAT, N)), _full((HEADS, N, DHE)),
                _full((LAT, LAT)), _full((1, LAT)), _full((1, LAT))]
    if fuse_w is None:
        return pl.pallas_call(
            _g_attn_body,
            grid=(N // BLKG,),
            in_specs=in_specs,
            out_specs=_rows((BLKG, LAT)),
            out_shape=out,
            compiler_params=_PARALLEL,
        )(s, q, kt, v, wo, gamma, beta)
    return pl.pallas_call(
        _g_attn_body_fused,
        grid=(N // BLKG,),
        in_specs=in_specs + [_full((LAT, LAT))] * 3,
        out_specs=[_rows((BLKG, LAT)), _rows((BLKG, LAT)),
                   pl.BlockSpec((LAT, BLKG), lambda i: (0, i)),
                   pl.BlockSpec((HEADS, BLKG, DHE), lambda i: (0, i, 0))],
        out_shape=[out, out,
                   jax.ShapeDtypeStruct((LAT, N), jnp.float32),
                   jax.ShapeDtypeStruct((HEADS, N, DHE), jnp.float32)],
        compiler_params=_PARALLEL,
    )(s, q, kt, v, wo, gamma, beta, *fuse_w)


def _head_body(tf, w0, b0, w1, b1, wmu, bmu, wsig, bsig, muo, sigo):
    m = jnp.mean(tf[...], axis=0, keepdims=True)
    h = jnp.maximum(jnp.dot(m, w0[...], preferred_element_type=jnp.float32)
                    + b0[...], 0.0)
    h = jnp.dot(h, w1[...], preferred_element_type=jnp.float32) + b1[...]
    muo[...] = jnp.dot(h, wmu[...], preferred_element_type=jnp.float32) + bmu[...]
    z = jnp.dot(h, wsig[...], preferred_element_type=jnp.float32) + bsig[...]
    sigo[...] = 0.1 + 0.9 * jax.nn.sigmoid(z)


def _head(t, w0, b0, w1, b1, wmu, bmu, wsig, bsig):
    out = jax.ShapeDtypeStruct((1, LAT), jnp.float32)
    return pl.pallas_call(
        _head_body,
        grid=(1,),
        in_specs=[_full((N, LAT))] + [_full((LAT, LAT)), _full((1, LAT))] * 4,
        out_specs=[_full((1, LAT))] * 2,
        out_shape=[out, out],
    )(t, w0, b0, w1, b1, wmu, bmu, wsig, bsig)


def kernel(x, y, task_labels, set_W0, set_b0, set_W1, set_b1,
           pt_Wq, pt_Wk, pt_Wv, pt_Wo, pt_gamma, pt_beta,
           g_Wq, g_Wk, g_Wv, g_Wo, g_gamma, g_beta,
           am_W0, am_b0, am_W1, am_b1, am_Wmu, am_bmu, am_Wsig, am_bsig):
    r = lambda b: b.reshape(1, LAT)

    lab = task_labels.astype(jnp.int32)
    tids = jnp.arange(TASKS, dtype=jnp.int32)
    oh = (lab[:, None] == tids[None, :]).astype(jnp.int32)
    counts = oh.sum(0)
    ends = jnp.cumsum(counts)
    starts = ends - counts
    cc = jnp.cumsum(oh, axis=0)
    rank = ((oh * starts[None, :]).sum(1) + (oh * cc).sum(1) - 1
            ).astype(jnp.int32)
    pos = jnp.arange(N, dtype=jnp.int32)
    lab_sorted = (pos[:, None] >= ends[None, :]).sum(1).astype(jnp.int32)
    lab_col = lab_sorted.reshape(N, 1)
    lab_chunks = lab_sorted.reshape(NCH, 1, CH)
    tfl = lab_sorted[::BLK]
    tfh = lab_sorted[BLK - 1::BLK]
    ohl = (tfl[:, None] == tids[None, :]).astype(jnp.int32)
    ohh = (tfh[:, None] == tids[None, :]).astype(jnp.int32)
    kstart = (ohl * starts[None, :]).sum(1)
    kend = (ohh * ends[None, :]).sum(1)
    blo = (kstart // CH).astype(jnp.int32)
    bhi = ((kend - 1) // CH).astype(jnp.int32)
    idx2d = rank.reshape(_NW, BPW)

    s = _set_mlp(x, y, set_W0[:x.shape[1]], set_W0[x.shape[1]:],
                 r(set_b0), set_W1, r(set_b1))

    sl = _sc_permute(s, idx2d, "scatter")
    sl = _pt_attn_layer(sl, lab_col, lab_chunks, tfl, tfh, blo, bhi,
                        pt_Wq[:, 0], pt_Wk[:, 0], pt_Wv[:, 0],
                        pt_Wo[:, 0], pt_gamma[:, 0], pt_beta[:, 0])
    sl, gq, gkt, gv = _pt_attn_layer(
        sl, lab_col, lab_chunks, tfl, tfh, blo, bhi,
        pt_Wq[:, 1], pt_Wk[:, 1], pt_Wv[:, 1],
        pt_Wo[:, 1], pt_gamma[:, 1], pt_beta[:, 1],
        fuse_w=(g_Wq[0], g_Wk[0], g_Wv[0]))

    t, gq2, gkt2, gv2 = _g_attn_call(
        sl, gq, gkt, gv, g_Wo[0], r(g_gamma[0]), r(g_beta[0]),
        fuse_w=(g_Wq[1], g_Wk[1], g_Wv[1]))
    t = _g_attn_call(t, gq2, gkt2, gv2, g_Wo[1], r(g_gamma[1]), r(g_beta[1]))

    mu, sig = _head(t, am_W0, r(am_b0), am_W1, r(am_b1),
                    am_Wmu, r(am_bmu), am_Wsig, r(am_bsig))
    s_local = _sc_permute(sl, idx2d, "gather")
    temp = _sc_permute(t, idx2d, "gather")
    return mu.reshape(LAT), sig.reshape(LAT), s_local, temp
